# baseline (speedup 1.0000x reference)
import functools

import jax
import jax.numpy as jnp
from jax import lax
from jax.experimental import pallas as pl
from jax.experimental.pallas import tpu as pltpu
from jax.experimental.pallas import tpu_sc as plsc

BATCH = 4096
EMB = 64
NC = 2
NS = 16
NW = NC * NS
B_PER_W = BATCH // NW


def _gather_body(user_tab, item_tab, uidx, iidx, uout, iout,
                 uidx_v, iidx_v, urows, irows, usem, isem):
    wid = lax.axis_index("s") * NC + lax.axis_index("c")
    base = wid * B_PER_W
    pltpu.sync_copy(uidx.at[pl.ds(base, B_PER_W)], uidx_v)
    pltpu.sync_copy(iidx.at[pl.ds(base, B_PER_W)], iidx_v)
    ucp = pltpu.async_copy(user_tab.at[uidx_v], urows, usem)
    icp = pltpu.async_copy(item_tab.at[iidx_v], irows, isem)
    ucp.wait()
    icp.wait()
    pltpu.sync_copy(urows, uout.at[pl.ds(base, B_PER_W)])
    pltpu.sync_copy(irows, iout.at[pl.ds(base, B_PER_W)])


@functools.partial(jax.jit, static_argnums=())
def _sc_gather(user, item, user_table, item_table):
    mesh = plsc.VectorSubcoreMesh(core_axis_name="c", subcore_axis_name="s",
                                  num_cores=NC, num_subcores=NS)
    return pl.kernel(
        _gather_body,
        out_type=(
            jax.ShapeDtypeStruct((BATCH, EMB), jnp.float32),
            jax.ShapeDtypeStruct((BATCH, EMB), jnp.float32),
        ),
        mesh=mesh,
        scratch_types=[
            pltpu.VMEM((B_PER_W,), jnp.int32),
            pltpu.VMEM((B_PER_W,), jnp.int32),
            pltpu.VMEM((B_PER_W, EMB), jnp.float32),
            pltpu.VMEM((B_PER_W, EMB), jnp.float32),
            pltpu.SemaphoreType.DMA,
            pltpu.SemaphoreType.DMA,
        ],
        compiler_params=pltpu.CompilerParams(use_tc_tiling_on_sc=False),
    )(user_table, item_table, user, item)


def _dense_body(ue_ref, ie_ref, w1u_ref, w1i_ref, b1_ref, w2_ref, b2_ref,
                w3_ref, b3_ref, gmf_w_ref, wf_h_ref, fbias_ref, out_ref):
    ue = ue_ref[...]
    ie = ie_ref[...]
    h = jnp.maximum(
        jnp.dot(ue, w1u_ref[...], preferred_element_type=jnp.float32)
        + jnp.dot(ie, w1i_ref[...], preferred_element_type=jnp.float32)
        + b1_ref[...], 0.0)
    h = jnp.maximum(
        jnp.dot(h, w2_ref[...], preferred_element_type=jnp.float32)
        + b2_ref[...], 0.0)
    h = jnp.maximum(
        jnp.dot(h, w3_ref[...], preferred_element_type=jnp.float32)
        + b3_ref[...], 0.0)
    gmf = jnp.sum(ue * ie * gmf_w_ref[...], axis=1, keepdims=True)
    z = (jnp.sum(h * wf_h_ref[...], axis=1, keepdims=True)
         + gmf * fbias_ref[0, 0] + fbias_ref[0, 1])
    out_ref[...] = 1.0 / (1.0 + jnp.exp(-z))


def _tc_dense(ue, ie, w1u, w1i, b1, w2, b2, w3, b3, gmf_w_row, wf_h_row, fbias):
    return pl.pallas_call(
        _dense_body,
        out_shape=jax.ShapeDtypeStruct((BATCH, 1), jnp.float32),
    )(ue, ie, w1u, w1i, b1, w2, b2, w3, b3, gmf_w_row, wf_h_row, fbias)


def kernel(user, item, user_table, item_table, gmf_W, gmf_b,
           fc1_W, fc1_b, fc2_W, fc2_b, fc3_W, fc3_b, final_W, final_b):
    ue, ie = _sc_gather(user.astype(jnp.int32), item.astype(jnp.int32),
                        user_table, item_table)
    w1u = fc1_W[:EMB]
    w1i = fc1_W[EMB:]
    b1 = fc1_b.reshape(1, -1)
    b2 = fc2_b.reshape(1, -1)
    b3 = fc3_b.reshape(1, -1)
    gmf_w_row = gmf_W.reshape(1, EMB)
    wf_h_row = final_W[1:, 0].reshape(1, 32)
    w0 = final_W[0, 0]
    fbias = jnp.stack([w0, final_b[0] + gmf_b[0] * w0]).reshape(1, 2)
    out = _tc_dense(ue, ie, w1u, w1i, b1, w2=fc2_W, b2=b2, w3=fc3_W, b3=b3,
                    gmf_w_row=gmf_w_row, wf_h_row=wf_h_row, fbias=fbias)
    return out.reshape(BATCH)


# speedup vs baseline: 1.6693x; 1.6693x over previous
import functools

import jax
import jax.numpy as jnp
from jax import lax
from jax.experimental import pallas as pl
from jax.experimental.pallas import tpu as pltpu
from jax.experimental.pallas import tpu_sc as plsc

BATCH = 4096
EMB = 64
NC = 2
NS = 16
NW = NC * NS
B_PER_W = BATCH // NW


def _gather_body(user_tab, item_tab, uidx, iidx, uout, iout,
                 uidx_v, iidx_v, urows, irows, usem, isem):
    wid = lax.axis_index("s") * NC + lax.axis_index("c")
    base = wid * B_PER_W
    pltpu.sync_copy(uidx.at[pl.ds(base, B_PER_W)], uidx_v)
    pltpu.sync_copy(iidx.at[pl.ds(base, B_PER_W)], iidx_v)

    lane = lax.broadcasted_iota(jnp.int32, (16,), 0)

    def fire(c, idx_v, tab, rows, sem):
        chunk = idx_v[pl.ds(c * 16, 16)]
        for j in range(16):
            row = jnp.sum(jnp.where(lane == j, chunk, 0))
            pltpu.async_copy(tab.at[pl.ds(row, 1)],
                             rows.at[pl.ds(c * 16 + j, 1)], sem)

    def ubody(c, carry):
        fire(c, uidx_v, user_tab, urows, usem)
        return carry

    def ibody(c, carry):
        fire(c, iidx_v, item_tab, irows, isem)
        return carry

    lax.fori_loop(0, B_PER_W // 16, ubody, 0, unroll=True)
    lax.fori_loop(0, B_PER_W // 16, ibody, 0, unroll=True)
    pltpu.make_async_copy(user_tab.at[pl.ds(0, B_PER_W)], urows, usem).wait()
    pltpu.make_async_copy(item_tab.at[pl.ds(0, B_PER_W)], irows, isem).wait()
    pltpu.sync_copy(urows, uout.at[pl.ds(base, B_PER_W)])
    pltpu.sync_copy(irows, iout.at[pl.ds(base, B_PER_W)])


@functools.partial(jax.jit, static_argnums=())
def _sc_gather(user, item, user_table, item_table):
    mesh = plsc.VectorSubcoreMesh(core_axis_name="c", subcore_axis_name="s",
                                  num_cores=NC, num_subcores=NS)
    return pl.kernel(
        _gather_body,
        out_type=(
            jax.ShapeDtypeStruct((BATCH, EMB), jnp.float32),
            jax.ShapeDtypeStruct((BATCH, EMB), jnp.float32),
        ),
        mesh=mesh,
        scratch_types=[
            pltpu.VMEM((B_PER_W,), jnp.int32),
            pltpu.VMEM((B_PER_W,), jnp.int32),
            pltpu.VMEM((B_PER_W, EMB), jnp.float32),
            pltpu.VMEM((B_PER_W, EMB), jnp.float32),
            pltpu.SemaphoreType.DMA,
            pltpu.SemaphoreType.DMA,
        ],
        compiler_params=pltpu.CompilerParams(needs_layout_passes=False),
    )(user_table, item_table, user, item)


def _dense_body(ue_ref, ie_ref, w1u_ref, w1i_ref, b1_ref, w2_ref, b2_ref,
                w3_ref, b3_ref, gmf_w_ref, wf_h_ref, fbias_ref, out_ref):
    ue = ue_ref[...]
    ie = ie_ref[...]
    h = jnp.maximum(
        jnp.dot(ue, w1u_ref[...], preferred_element_type=jnp.float32)
        + jnp.dot(ie, w1i_ref[...], preferred_element_type=jnp.float32)
        + b1_ref[...], 0.0)
    h = jnp.maximum(
        jnp.dot(h, w2_ref[...], preferred_element_type=jnp.float32)
        + b2_ref[...], 0.0)
    h = jnp.maximum(
        jnp.dot(h, w3_ref[...], preferred_element_type=jnp.float32)
        + b3_ref[...], 0.0)
    gmf = jnp.sum(ue * ie * gmf_w_ref[...], axis=1, keepdims=True)
    z = (jnp.sum(h * wf_h_ref[...], axis=1, keepdims=True)
         + gmf * fbias_ref[0, 0] + fbias_ref[0, 1])
    out_ref[...] = 1.0 / (1.0 + jnp.exp(-z))


def _tc_dense(ue, ie, w1u, w1i, b1, w2, b2, w3, b3, gmf_w_row, wf_h_row, fbias):
    return pl.pallas_call(
        _dense_body,
        out_shape=jax.ShapeDtypeStruct((BATCH, 1), jnp.float32),
    )(ue, ie, w1u, w1i, b1, w2, b2, w3, b3, gmf_w_row, wf_h_row, fbias)


def kernel(user, item, user_table, item_table, gmf_W, gmf_b,
           fc1_W, fc1_b, fc2_W, fc2_b, fc3_W, fc3_b, final_W, final_b):
    ue, ie = _sc_gather(user.astype(jnp.int32), item.astype(jnp.int32),
                        user_table, item_table)
    w1u = fc1_W[:EMB]
    w1i = fc1_W[EMB:]
    b1 = fc1_b.reshape(1, -1)
    b2 = fc2_b.reshape(1, -1)
    b3 = fc3_b.reshape(1, -1)
    gmf_w_row = gmf_W.reshape(1, EMB)
    wf_h_row = final_W[1:, 0].reshape(1, 32)
    w0 = final_W[0, 0]
    fbias = jnp.stack([w0, final_b[0] + gmf_b[0] * w0]).reshape(1, 2)
    out = _tc_dense(ue, ie, w1u, w1i, b1, w2=fc2_W, b2=b2, w3=fc3_W, b3=b3,
                    gmf_w_row=gmf_w_row, wf_h_row=wf_h_row, fbias=fbias)
    return out.reshape(BATCH)
